# baseline (device time: 61337 ns/iter reference)
import jax
import jax.numpy as jnp
from jax import lax
from jax.experimental import pallas as pl
from jax.experimental.pallas import tpu as pltpu

N_DEV = 4


def _gelu(z):
    return 0.5 * z * (1.0 + jnp.tanh(0.7978845608 * (z + 0.044715 * z * z * z)))


def kernel(A, B):
    m, k_per = A.shape
    _, n = B.shape
    m_chunk = m // N_DEV
    n_half = n // 2

    def body(a_ref, b_ref, out_ref, a16, b16, partial_ref,
             srs_r, srs_l, rs_r, rs_l, gsb_r, gsb_l, ag_r, ag_l,
             rs_r_send, rs_r_recv, rs_l_send, rs_l_recv,
             ag_r_send, ag_r_recv, ag_l_send, ag_l_recv):
        my = lax.axis_index("i")
        left = lax.rem(my + N_DEV - 1, N_DEV)
        right = lax.rem(my + 1, N_DEV)

        def mod4(x):
            return lax.rem(x + 2 * N_DEV, N_DEV)

        def rows(c):
            return pl.ds(c * m_chunk, m_chunk)

        lo = pl.ds(0, n_half)
        hi = pl.ds(n_half, n_half)

        def mm(c, col_lo):
            a_chunk = a16[rows(c), :]
            b_half = b16[:, 0:n_half] if col_lo else b16[:, n_half:n]
            partial_ref[rows(c), lo if col_lo else hi] = jnp.dot(
                a_chunk, b_half, preferred_element_type=jnp.float32
            ).astype(jnp.bfloat16)

        a16[:, :] = a_ref[:, :].astype(jnp.bfloat16)
        b16[:, :] = b_ref[:, :].astype(jnp.bfloat16)

        barrier_sem = pltpu.get_barrier_semaphore()
        for nbr in (left, right):
            pl.semaphore_signal(
                barrier_sem, inc=1,
                device_id=(nbr,), device_id_type=pl.DeviceIdType.MESH,
            )
        pl.semaphore_wait(barrier_sem, 2)

        mm(my, True)
        mm(my, False)
        srs_r[0, :, :] = partial_ref[rows(my), lo]
        srs_l[0, :, :] = partial_ref[rows(my), hi]

        for s in range(N_DEV - 1):
            r = pltpu.make_async_remote_copy(
                src_ref=srs_r.at[s], dst_ref=rs_r.at[s],
                send_sem=rs_r_send.at[s], recv_sem=rs_r_recv.at[s],
                device_id=(right,), device_id_type=pl.DeviceIdType.MESH,
            )
            l = pltpu.make_async_remote_copy(
                src_ref=srs_l.at[s], dst_ref=rs_l.at[s],
                send_sem=rs_l_send.at[s], recv_sem=rs_l_recv.at[s],
                device_id=(left,), device_id_type=pl.DeviceIdType.MESH,
            )
            r.start()
            l.start()
            c_r = mod4(my - s - 1)
            c_l = mod4(my + s + 1)
            mm(c_r, True)
            mm(c_l, False)
            r.wait()
            l.wait()
            if s < N_DEV - 2:
                srs_r[s + 1, :, :] = rs_r[s, :, :] + partial_ref[rows(c_r), lo]
                srs_l[s + 1, :, :] = rs_l[s, :, :] + partial_ref[rows(c_l), hi]
            else:
                g_r = _gelu(rs_r[s, :, :].astype(jnp.float32)
                            + partial_ref[rows(c_r), lo].astype(jnp.float32))
                g_l = _gelu(rs_l[s, :, :].astype(jnp.float32)
                            + partial_ref[rows(c_l), hi].astype(jnp.float32))
                out_ref[rows(mod4(my + 1)), lo] = g_r
                out_ref[rows(mod4(my - 1)), hi] = g_l
                gsb_r[:, :] = g_r.astype(jnp.bfloat16)
                gsb_l[:, :] = g_l.astype(jnp.bfloat16)

        n_q = n_half // 2
        q0 = pl.ds(0, n_q)
        q1 = pl.ds(n_q, n_q)
        descs = []
        for h in range(N_DEV - 1):
            hop = []
            for (stage, ring_buf, sems_s, sems_r, dev) in (
                (gsb_r, ag_r, ag_r_send, ag_r_recv, right),
                (gsb_l, ag_l, ag_l_send, ag_l_recv, left),
            ):
                for qs, sem_base in ((q0, 0), (q1, N_DEV - 1)):
                    hop.append(pltpu.make_async_remote_copy(
                        src_ref=(stage.at[:, qs] if h == 0
                                 else ring_buf.at[h - 1, :, qs]),
                        dst_ref=ring_buf.at[h, :, qs],
                        send_sem=sems_s.at[sem_base + h],
                        recv_sem=sems_r.at[sem_base + h],
                        device_id=(dev,), device_id_type=pl.DeviceIdType.MESH,
                    ))
            descs.append(hop)
        for d in descs[0]:
            d.start()
        for h in range(N_DEV - 1):
            for qi, d in enumerate(descs[h]):
                d.wait_recv()
                if h < N_DEV - 2:
                    descs[h + 1][qi].start()
            out_ref[rows(mod4(my - h)), lo] = ag_r[h, :, :].astype(jnp.float32)
            out_ref[rows(mod4(my + h)), hi] = ag_l[h, :, :].astype(jnp.float32)
        for hop in descs:
            for d in hop:
                d.wait_send()

    comm_shape = (N_DEV - 1, m_chunk, n_half)
    return pl.pallas_call(
        body,
        out_shape=jax.ShapeDtypeStruct((m, n), jnp.float32),
        in_specs=[
            pl.BlockSpec(memory_space=pltpu.VMEM),
            pl.BlockSpec(memory_space=pltpu.VMEM),
        ],
        out_specs=pl.BlockSpec(memory_space=pltpu.VMEM),
        scratch_shapes=[
            pltpu.VMEM((m, k_per), jnp.bfloat16),
            pltpu.VMEM((k_per, n), jnp.bfloat16),
            pltpu.VMEM((m, n), jnp.bfloat16),
            pltpu.VMEM(comm_shape, jnp.bfloat16),
            pltpu.VMEM(comm_shape, jnp.bfloat16),
            pltpu.VMEM(comm_shape, jnp.bfloat16),
            pltpu.VMEM(comm_shape, jnp.bfloat16),
            pltpu.VMEM((m_chunk, n_half), jnp.bfloat16),
            pltpu.VMEM((m_chunk, n_half), jnp.bfloat16),
            pltpu.VMEM(comm_shape, jnp.bfloat16),
            pltpu.VMEM(comm_shape, jnp.bfloat16),
            pltpu.SemaphoreType.DMA((N_DEV - 1,)),
            pltpu.SemaphoreType.DMA((N_DEV - 1,)),
            pltpu.SemaphoreType.DMA((N_DEV - 1,)),
            pltpu.SemaphoreType.DMA((N_DEV - 1,)),
            pltpu.SemaphoreType.DMA((2 * (N_DEV - 1),)),
            pltpu.SemaphoreType.DMA((2 * (N_DEV - 1),)),
            pltpu.SemaphoreType.DMA((2 * (N_DEV - 1),)),
            pltpu.SemaphoreType.DMA((2 * (N_DEV - 1),)),
        ],
        compiler_params=pltpu.CompilerParams(collective_id=0),
    )(A, B)


# device time: 53486 ns/iter; 1.1468x vs baseline; 1.1468x over previous
import jax
import jax.numpy as jnp
from jax import lax
from jax.experimental import pallas as pl
from jax.experimental.pallas import tpu as pltpu

N_DEV = 4
N_STEP = N_DEV - 1
N_LANE = 4


def _gelu(z):
    return 0.5 * z * (1.0 + jnp.tanh(0.7978845608 * (z + 0.044715 * z * z * z)))


def kernel(A, B):
    m, k_per = A.shape
    _, n = B.shape
    m_chunk = m // N_DEV
    n_q = n // N_LANE

    def body(a_ref, b_ref, out_ref, a16, b16, partial_ref,
             srs, rsb, gsb, agb, rs_send, rs_recv, ag_send, ag_recv):
        my = lax.axis_index("i")
        left = lax.rem(my + N_DEV - 1, N_DEV)
        right = lax.rem(my + 1, N_DEV)

        def mod4(x):
            return lax.rem(x + 2 * N_DEV, N_DEV)

        def rows(c):
            return pl.ds(c * m_chunk, m_chunk)

        lanes = [(0, 1), (2, -1), (1, 1), (3, -1)]

        def cq(lane):
            return pl.ds(lanes[lane][0] * n_q, n_q)

        def dev(lane):
            return right if lanes[lane][1] > 0 else left

        def c_recv(lane, s):
            return mod4(my - lanes[lane][1] * (s + 1))

        def mm(c, col_lo):
            a_chunk = a16[rows(c), :]
            b_half = b16[:, 0:2 * n_q] if col_lo else b16[:, 2 * n_q:n]
            partial_ref[rows(c), pl.ds(0, 2 * n_q) if col_lo
                        else pl.ds(2 * n_q, 2 * n_q)] = jnp.dot(
                a_chunk, b_half, preferred_element_type=jnp.float32
            ).astype(jnp.bfloat16)

        a16[:, :] = a_ref[:, :].astype(jnp.bfloat16)
        b16[:, :] = b_ref[:, :].astype(jnp.bfloat16)

        barrier_sem = pltpu.get_barrier_semaphore()
        for nbr in (left, right):
            pl.semaphore_signal(
                barrier_sem, inc=1,
                device_id=(nbr,), device_id_type=pl.DeviceIdType.MESH,
            )
        pl.semaphore_wait(barrier_sem, 2)

        def sem_i(lane, s):
            return lane * N_STEP + s

        rs_d = [[pltpu.make_async_remote_copy(
                    src_ref=srs.at[k, s], dst_ref=rsb.at[k, s],
                    send_sem=rs_send.at[sem_i(k, s)],
                    recv_sem=rs_recv.at[sem_i(k, s)],
                    device_id=(dev(k),), device_id_type=pl.DeviceIdType.MESH)
                 for s in range(N_STEP)] for k in range(N_LANE)]
        ag_d = [[pltpu.make_async_remote_copy(
                    src_ref=(gsb.at[k] if h == 0 else agb.at[k, h - 1]),
                    dst_ref=agb.at[k, h],
                    send_sem=ag_send.at[sem_i(k, h)],
                    recv_sem=ag_recv.at[sem_i(k, h)],
                    device_id=(dev(k),), device_id_type=pl.DeviceIdType.MESH)
                 for h in range(N_STEP)] for k in range(N_LANE)]

        mm(my, True)
        mm(my, False)
        for k in range(N_LANE):
            srs[k, 0, :, :] = partial_ref[rows(my), cq(k)]
            rs_d[k][0].start()
        mm(mod4(my - 1), True)
        mm(mod4(my + 1), False)
        mm(mod4(my - 2), True)
        mm(mod4(my + 2), False)
        mm(mod4(my + 1), True)
        mm(mod4(my - 1), False)

        for s in range(N_STEP):
            for k in range(N_LANE):
                rs_d[k][s].wait()
                c = c_recv(k, s)
                if s < N_STEP - 1:
                    srs[k, s + 1, :, :] = (
                        rsb[k, s, :, :] + partial_ref[rows(c), cq(k)]
                    )
                    rs_d[k][s + 1].start()
                else:
                    g = _gelu(rsb[k, s, :, :].astype(jnp.float32)
                              + partial_ref[rows(c), cq(k)].astype(jnp.float32))
                    gsb[k, :, :] = g.astype(jnp.bfloat16)
                    ag_d[k][0].start()
                    out_ref[rows(mod4(my + lanes[k][1])), cq(k)] = g

        for h in range(N_STEP):
            for k in range(N_LANE):
                ag_d[k][h].wait_recv()
                if h < N_STEP - 1:
                    ag_d[k][h + 1].start()
            for k in range(N_LANE):
                out_ref[rows(mod4(my - lanes[k][1] * h)), cq(k)] = (
                    agb[k, h, :, :].astype(jnp.float32)
                )
        for k in range(N_LANE):
            for h in range(N_STEP):
                ag_d[k][h].wait_send()

    lane_shape = (N_LANE, N_STEP, m_chunk, n_q)
    n_sem = N_LANE * N_STEP
    return pl.pallas_call(
        body,
        out_shape=jax.ShapeDtypeStruct((m, n), jnp.float32),
        in_specs=[
            pl.BlockSpec(memory_space=pltpu.VMEM),
            pl.BlockSpec(memory_space=pltpu.VMEM),
        ],
        out_specs=pl.BlockSpec(memory_space=pltpu.VMEM),
        scratch_shapes=[
            pltpu.VMEM((m, k_per), jnp.bfloat16),
            pltpu.VMEM((k_per, n), jnp.bfloat16),
            pltpu.VMEM((m, n), jnp.bfloat16),
            pltpu.VMEM(lane_shape, jnp.bfloat16),
            pltpu.VMEM(lane_shape, jnp.bfloat16),
            pltpu.VMEM((N_LANE, m_chunk, n_q), jnp.bfloat16),
            pltpu.VMEM(lane_shape, jnp.bfloat16),
            pltpu.SemaphoreType.DMA((n_sem,)),
            pltpu.SemaphoreType.DMA((n_sem,)),
            pltpu.SemaphoreType.DMA((n_sem,)),
            pltpu.SemaphoreType.DMA((n_sem,)),
        ],
        compiler_params=pltpu.CompilerParams(collective_id=0),
    )(A, B)
